# baseline (speedup 1.0000x reference)
import functools
import math

import jax
import jax.numpy as jnp
from jax.experimental import pallas as pl
from jax.experimental.pallas import tpu as pltpu

B, S, D = 1, 2048, 768
H, DH = 12, 64
E, TOPK, DFF = 8, 2, 256
EPS = 1e-6
ROUTE_SCALE = 2.5
TB = 256
NTB = S // TB


def _qkv_body(x_ref, wqkv_ref, ln_ref, cos_ref, sin_ref, qkv_ref):
    xb = x_ref[...]
    rs = jax.lax.rsqrt(jnp.mean(xb * xb, axis=1, keepdims=True) + EPS)
    xn = (xb * rs * ln_ref[...]).astype(jnp.bfloat16)
    qkv = jnp.dot(xn, wqkv_ref[...], preferred_element_type=jnp.float32)
    cos = cos_ref[...]
    sin = sin_ref[...]
    scale = 1.0 / math.sqrt(DH)
    for h in range(H):
        for slot, base, sc in ((h, h * DH, scale), (H + h, D + h * DH, 1.0)):
            te = qkv[:, base:base + DH // 2]
            to = qkv[:, base + DH // 2:base + DH]
            qkv_ref[slot, :, :DH // 2] = ((te * cos - to * sin) * sc
                                          ).astype(jnp.bfloat16)
            qkv_ref[slot, :, DH // 2:] = ((to * cos + te * sin) * sc
                                          ).astype(jnp.bfloat16)
    for h in range(H):
        base = 2 * D + h * DH
        qkv_ref[2 * H + h, :, :] = qkv[:, base:base + DH].astype(jnp.bfloat16)


def _att_body(q_ref, k_ref, v_ref, o_ref):
    qb = pl.program_id(1)
    q = q_ref[0]
    rows = qb * TB + jax.lax.broadcasted_iota(jnp.int32, (TB, TB), 0)

    def step(j, carry):
        m, l, acc = carry
        off = pl.multiple_of(j * TB, TB)
        k = k_ref[0, pl.ds(off, TB), :]
        s = jax.lax.dot_general(q, k, (((1,), (1,)), ((), ())),
                                preferred_element_type=jnp.float32)
        cols = j * TB + jax.lax.broadcasted_iota(jnp.int32, (TB, TB), 1)
        s = jnp.where(cols <= rows, s, -jnp.inf)
        m_new = jnp.maximum(m, jnp.max(s, axis=1, keepdims=True))
        p = jnp.exp(s - m_new)
        corr = jnp.exp(m - m_new)
        l = l * corr + jnp.sum(p, axis=1, keepdims=True)
        v = v_ref[0, pl.ds(off, TB), :]
        acc = acc * corr + jnp.dot(p.astype(jnp.bfloat16), v,
                                   preferred_element_type=jnp.float32)
        return m_new, l, acc

    m0 = jnp.full((TB, 1), -jnp.inf, jnp.float32)
    l0 = jnp.zeros((TB, 1), jnp.float32)
    acc0 = jnp.zeros((TB, DH), jnp.float32)
    m, l, acc = jax.lax.fori_loop(0, qb + 1, step, (m0, l0, acc0))
    o_ref[0] = acc / l


def _gate_body(a_ref, x_ref, wo_ref, ln_ref, gwt_ref, eids_ref,
               x1_ref, xn2_ref, comb_ref):
    a2 = jnp.concatenate([a_ref[h] for h in range(H)], axis=1)
    a = jnp.dot(a2.astype(jnp.bfloat16), wo_ref[...],
                preferred_element_type=jnp.float32)

    x1 = a + x_ref[...]
    x1_ref[...] = x1
    rs = jax.lax.rsqrt(jnp.mean(x1 * x1, axis=1, keepdims=True) + EPS)
    xn2 = x1 * rs * ln_ref[...]
    xn2_ref[...] = xn2
    sc = jax.nn.sigmoid(jnp.dot(xn2, gwt_ref[...],
                                preferred_element_type=jnp.float32))
    w0 = jnp.zeros((TB, 1), jnp.float32)
    w1_ = jnp.zeros((TB, 1), jnp.float32)
    e0 = eids_ref[:, 0:1]
    e1 = eids_ref[:, 1:2]
    for e in range(E):
        w0 = jnp.where(e0 == e, sc[:, e:e + 1], w0)
        w1_ = jnp.where(e1 == e, sc[:, e:e + 1], w1_)
    scale = ROUTE_SCALE / (w0 + w1_ + 1e-20)
    w0 = w0 * scale
    w1_ = w1_ * scale
    comb = jnp.zeros((TB, E), jnp.float32)
    for e in range(E):
        comb = comb + jnp.where(e0 == e, w0, 0.0) * (jax.lax.broadcasted_iota(jnp.int32, (TB, E), 1) == e)
        comb = comb + jnp.where(e1 == e, w1_, 0.0) * (jax.lax.broadcasted_iota(jnp.int32, (TB, E), 1) == e)
    comb_ref[...] = comb


def _moe_body(xn2_ref, w1_ref, w3_ref, w2_ref, comb_ref, x1_ref, out_ref):
    e = pl.program_id(1)

    @pl.when(e == 0)
    def _():
        out_ref[...] = x1_ref[...]

    xn2 = xn2_ref[...].astype(jnp.bfloat16)
    h = jnp.dot(xn2, w1_ref[0], preferred_element_type=jnp.float32)
    g = jnp.dot(xn2, w3_ref[0], preferred_element_type=jnp.float32)
    he = (h * jax.nn.sigmoid(h) * g).astype(jnp.bfloat16)
    o = jnp.dot(he, w2_ref[0], preferred_element_type=jnp.float32)
    onehot = (jax.lax.broadcasted_iota(jnp.int32, (E, 1), 0) == e
              ).astype(jnp.float32)
    ccol = jnp.dot(comb_ref[...], onehot, preferred_element_type=jnp.float32)
    out_ref[...] += ccol * o


def kernel(x, freqs_cis, input_ids, w_ln1, w_ln2, wq, wk, wv, wo, gate_w,
           tid2eid, w1, w2, w3):
    x2d = x.reshape(S, D)
    perm = jnp.concatenate([jnp.arange(0, DH, 2), jnp.arange(1, DH, 2)])
    head_perm = (jnp.arange(H)[:, None] * DH + perm[None, :]).reshape(-1)
    wq_p = wq[:, head_perm]
    wk_p = wk[:, head_perm]
    wqkv = jnp.concatenate([wq_p, wk_p, wv], axis=1).astype(jnp.bfloat16)
    cos = freqs_cis[:, :, 0]
    sin = freqs_cis[:, :, 1]

    qkv = pl.pallas_call(
        _qkv_body,
        grid=(NTB,),
        in_specs=[
            pl.BlockSpec((TB, D), lambda i: (i, 0)),
            pl.BlockSpec((D, 3 * D), lambda i: (0, 0)),
            pl.BlockSpec((1, D), lambda i: (0, 0)),
            pl.BlockSpec((TB, DH // 2), lambda i: (i, 0)),
            pl.BlockSpec((TB, DH // 2), lambda i: (i, 0)),
        ],
        out_specs=pl.BlockSpec((3 * H, TB, DH), lambda i: (0, i, 0)),
        out_shape=jax.ShapeDtypeStruct((3 * H, S, DH), jnp.bfloat16),
    )(x2d, wqkv, w_ln1.reshape(1, D), cos, sin)

    attn = pl.pallas_call(
        _att_body,
        grid=(H, NTB),
        in_specs=[
            pl.BlockSpec((1, TB, DH), lambda h, i: (h, i, 0)),
            pl.BlockSpec((1, S, DH), lambda h, i: (h + H, 0, 0)),
            pl.BlockSpec((1, S, DH), lambda h, i: (h + 2 * H, 0, 0)),
        ],
        out_specs=pl.BlockSpec((1, TB, DH), lambda h, i: (h, i, 0)),
        out_shape=jax.ShapeDtypeStruct((H, S, DH), jnp.float32),
    )(qkv, qkv, qkv)

    eids = jnp.take(tid2eid, input_ids.reshape(-1), axis=0).astype(jnp.int32)

    x1, xn2, comb = pl.pallas_call(
        _gate_body,
        grid=(NTB,),
        in_specs=[
            pl.BlockSpec((H, TB, DH), lambda i: (0, i, 0)),
            pl.BlockSpec((TB, D), lambda i: (i, 0)),
            pl.BlockSpec((D, D), lambda i: (0, 0)),
            pl.BlockSpec((1, D), lambda i: (0, 0)),
            pl.BlockSpec((D, E), lambda i: (0, 0)),
            pl.BlockSpec((TB, TOPK), lambda i: (i, 0)),
        ],
        out_specs=[
            pl.BlockSpec((TB, D), lambda i: (i, 0)),
            pl.BlockSpec((TB, D), lambda i: (i, 0)),
            pl.BlockSpec((TB, E), lambda i: (i, 0)),
        ],
        out_shape=[
            jax.ShapeDtypeStruct((S, D), jnp.float32),
            jax.ShapeDtypeStruct((S, D), jnp.float32),
            jax.ShapeDtypeStruct((S, E), jnp.float32),
        ],
    )(attn, x2d, wo.astype(jnp.bfloat16), w_ln2.reshape(1, D), gate_w.T, eids)

    out = pl.pallas_call(
        _moe_body,
        grid=(NTB, E),
        in_specs=[
            pl.BlockSpec((TB, D), lambda i, e: (i, 0)),
            pl.BlockSpec((1, D, DFF), lambda i, e: (e, 0, 0)),
            pl.BlockSpec((1, D, DFF), lambda i, e: (e, 0, 0)),
            pl.BlockSpec((1, DFF, D), lambda i, e: (e, 0, 0)),
            pl.BlockSpec((TB, E), lambda i, e: (i, 0)),
            pl.BlockSpec((TB, D), lambda i, e: (i, 0)),
        ],
        out_specs=pl.BlockSpec((TB, D), lambda i, e: (i, 0)),
        out_shape=jax.ShapeDtypeStruct((S, D), jnp.float32),
        compiler_params=pltpu.CompilerParams(
            dimension_semantics=("parallel", "arbitrary")),
    )(xn2, w1.astype(jnp.bfloat16), w3.astype(jnp.bfloat16),
      w2.astype(jnp.bfloat16), comb, x1)

    return out.reshape(B, S, D)


# speedup vs baseline: 1.5848x; 1.5848x over previous
import functools
import math

import jax
import jax.numpy as jnp
from jax.experimental import pallas as pl
from jax.experimental.pallas import tpu as pltpu

B, S, D = 1, 2048, 768
H, DH = 12, 64
E, TOPK, DFF = 8, 2, 256
EPS = 1e-6
ROUTE_SCALE = 2.5
TB = 256
NTB = S // TB


def _qkv_body(x_ref, wqkv_ref, ln_ref, cos_ref, sin_ref, qkv_ref):
    xb = x_ref[...]
    rs = jax.lax.rsqrt(jnp.mean(xb * xb, axis=1, keepdims=True) + EPS)
    xn = (xb * rs * ln_ref[...]).astype(jnp.bfloat16)
    qkv = jnp.dot(xn, wqkv_ref[...], preferred_element_type=jnp.float32)
    cos = cos_ref[...]
    sin = sin_ref[...]
    scale = 1.0 / math.sqrt(DH)
    for h in range(H):
        for slot, base, sc in ((h, h * DH, scale), (H + h, D + h * DH, 1.0)):
            te = qkv[:, base:base + DH // 2]
            to = qkv[:, base + DH // 2:base + DH]
            qkv_ref[slot, :, :DH // 2] = ((te * cos - to * sin) * sc
                                          ).astype(jnp.bfloat16)
            qkv_ref[slot, :, DH // 2:] = ((to * cos + te * sin) * sc
                                          ).astype(jnp.bfloat16)
    for h in range(H):
        base = 2 * D + h * DH
        qkv_ref[2 * H + h, :, :] = qkv[:, base:base + DH].astype(jnp.bfloat16)


def _att_body(q_ref, k_ref, v_ref, o_ref):
    qb = pl.program_id(1)
    q = q_ref[0]
    k = k_ref[0]
    s = jax.lax.dot_general(q, k, (((1,), (1,)), ((), ())),
                            preferred_element_type=jnp.float32)
    rows = qb * TB + jax.lax.broadcasted_iota(jnp.int32, (TB, S), 0)
    cols = jax.lax.broadcasted_iota(jnp.int32, (TB, S), 1)
    s = jnp.where(cols <= rows, s, -jnp.inf)
    m = jnp.max(s, axis=1, keepdims=True)
    p = jnp.exp(s - m)
    denom = jnp.sum(p, axis=1, keepdims=True)
    o = jnp.dot(p.astype(jnp.bfloat16), v_ref[0],
                preferred_element_type=jnp.float32)
    o_ref[0] = o / denom


def _gate_body(a_ref, x_ref, wo_ref, ln_ref, gwt_ref, eids_ref,
               x1_ref, xn2_ref, comb_ref):
    a2 = jnp.concatenate([a_ref[h] for h in range(H)], axis=1)
    a = jnp.dot(a2.astype(jnp.bfloat16), wo_ref[...].astype(jnp.bfloat16),
                preferred_element_type=jnp.float32)

    x1 = a + x_ref[...]
    x1_ref[...] = x1
    rs = jax.lax.rsqrt(jnp.mean(x1 * x1, axis=1, keepdims=True) + EPS)
    xn2 = x1 * rs * ln_ref[...]
    xn2_ref[...] = xn2.astype(jnp.bfloat16)
    sc = jax.nn.sigmoid(jnp.dot(xn2, gwt_ref[...],
                                preferred_element_type=jnp.float32))
    e0 = eids_ref[:, 0:1]
    e1 = eids_ref[:, 1:2]
    ee = jax.lax.broadcasted_iota(jnp.int32, (TB, E), 1)
    m0 = (ee == e0)
    m1 = (ee == e1)
    w0 = jnp.sum(jnp.where(m0, sc, 0.0), axis=1, keepdims=True)
    w1_ = jnp.sum(jnp.where(m1, sc, 0.0), axis=1, keepdims=True)
    scale = ROUTE_SCALE / (w0 + w1_ + 1e-20)
    comb_ref[...] = (jnp.where(m0, w0, 0.0) + jnp.where(m1, w1_, 0.0)) * scale


def _moe_body(xn2_ref, w1_ref, w3_ref, w2_ref, comb_ref, x1_ref, out_ref):
    e = pl.program_id(0)

    @pl.when(e == 0)
    def _():
        out_ref[...] = x1_ref[...]

    xn2 = xn2_ref[...]
    h = jnp.dot(xn2, w1_ref[0].astype(jnp.bfloat16),
                preferred_element_type=jnp.float32)
    g = jnp.dot(xn2, w3_ref[0].astype(jnp.bfloat16),
                preferred_element_type=jnp.float32)
    he = (h * jax.nn.sigmoid(h) * g).astype(jnp.bfloat16)
    o = jnp.dot(he, w2_ref[0].astype(jnp.bfloat16),
                preferred_element_type=jnp.float32)
    onehot = (jax.lax.broadcasted_iota(jnp.int32, (E, 1), 0) == e
              ).astype(jnp.float32)
    ccol = jnp.dot(comb_ref[...], onehot, preferred_element_type=jnp.float32)
    out_ref[...] += ccol * o


def kernel(x, freqs_cis, input_ids, w_ln1, w_ln2, wq, wk, wv, wo, gate_w,
           tid2eid, w1, w2, w3):
    x2d = x.reshape(S, D)
    perm = jnp.concatenate([jnp.arange(0, DH, 2), jnp.arange(1, DH, 2)])
    head_perm = (jnp.arange(H)[:, None] * DH + perm[None, :]).reshape(-1)
    wq_p = wq[:, head_perm]
    wk_p = wk[:, head_perm]
    wqkv = jnp.concatenate([wq_p, wk_p, wv], axis=1).astype(jnp.bfloat16)
    cos = freqs_cis[:, :, 0]
    sin = freqs_cis[:, :, 1]

    qkv = pl.pallas_call(
        _qkv_body,
        grid=(NTB,),
        in_specs=[
            pl.BlockSpec((TB, D), lambda i: (i, 0)),
            pl.BlockSpec((D, 3 * D), lambda i: (0, 0)),
            pl.BlockSpec((1, D), lambda i: (0, 0)),
            pl.BlockSpec((TB, DH // 2), lambda i: (i, 0)),
            pl.BlockSpec((TB, DH // 2), lambda i: (i, 0)),
        ],
        out_specs=pl.BlockSpec((3 * H, TB, DH), lambda i: (0, i, 0)),
        out_shape=jax.ShapeDtypeStruct((3 * H, S, DH), jnp.bfloat16),
    )(x2d, wqkv, w_ln1.reshape(1, D), cos, sin)

    attn = pl.pallas_call(
        _att_body,
        grid=(H, NTB),
        in_specs=[
            pl.BlockSpec((1, TB, DH), lambda h, i: (h, i, 0)),
            pl.BlockSpec((1, S, DH), lambda h, i: (h + H, 0, 0)),
            pl.BlockSpec((1, S, DH), lambda h, i: (h + 2 * H, 0, 0)),
        ],
        out_specs=pl.BlockSpec((1, TB, DH), lambda h, i: (h, i, 0)),
        out_shape=jax.ShapeDtypeStruct((H, S, DH), jnp.float32),
    )(qkv, qkv, qkv)

    eids = jnp.take(tid2eid, input_ids.reshape(-1), axis=0).astype(jnp.int32)

    x1, xn2, comb = pl.pallas_call(
        _gate_body,
        grid=(NTB,),
        in_specs=[
            pl.BlockSpec((H, TB, DH), lambda i: (0, i, 0)),
            pl.BlockSpec((TB, D), lambda i: (i, 0)),
            pl.BlockSpec((D, D), lambda i: (0, 0)),
            pl.BlockSpec((1, D), lambda i: (0, 0)),
            pl.BlockSpec((D, E), lambda i: (0, 0)),
            pl.BlockSpec((TB, TOPK), lambda i: (i, 0)),
        ],
        out_specs=[
            pl.BlockSpec((TB, D), lambda i: (i, 0)),
            pl.BlockSpec((TB, D), lambda i: (i, 0)),
            pl.BlockSpec((TB, E), lambda i: (i, 0)),
        ],
        out_shape=[
            jax.ShapeDtypeStruct((S, D), jnp.float32),
            jax.ShapeDtypeStruct((S, D), jnp.bfloat16),
            jax.ShapeDtypeStruct((S, E), jnp.float32),
        ],
    )(attn, x2d, wo, w_ln2.reshape(1, D), gate_w.T, eids)

    out = pl.pallas_call(
        _moe_body,
        grid=(E,),
        in_specs=[
            pl.BlockSpec((S, D), lambda e: (0, 0)),
            pl.BlockSpec((1, D, DFF), lambda e: (e, 0, 0)),
            pl.BlockSpec((1, D, DFF), lambda e: (e, 0, 0)),
            pl.BlockSpec((1, DFF, D), lambda e: (e, 0, 0)),
            pl.BlockSpec((S, E), lambda e: (0, 0)),
            pl.BlockSpec((S, D), lambda e: (0, 0)),
        ],
        out_specs=pl.BlockSpec((S, D), lambda e: (0, 0)),
        out_shape=jax.ShapeDtypeStruct((S, D), jnp.float32),
    )(xn2, w1, w3, w2, comb, x1)

    return out.reshape(B, S, D)


# speedup vs baseline: 1.8103x; 1.1423x over previous
import functools
import math

import jax
import jax.numpy as jnp
from jax.experimental import pallas as pl
from jax.experimental.pallas import tpu as pltpu

B, S, D = 1, 2048, 768
H, DH = 12, 64
E, TOPK, DFF = 8, 2, 256
EPS = 1e-6
ROUTE_SCALE = 2.5
TB = 256
NTB = S // TB


def _qkv_body(x_ref, wqkv_ref, ln_ref, cos_ref, sin_ref, qkv_ref):
    xb = x_ref[...]
    rs = jax.lax.rsqrt(jnp.mean(xb * xb, axis=1, keepdims=True) + EPS)
    xn = (xb * rs * ln_ref[...]).astype(jnp.bfloat16)
    qkv = jnp.dot(xn, wqkv_ref[...], preferred_element_type=jnp.float32)
    cos = cos_ref[...]
    sin = sin_ref[...]
    scale = 1.0 / math.sqrt(DH)
    for h in range(H):
        for slot, base, sc in ((h, h * DH, scale), (H + h, D + h * DH, 1.0)):
            te = qkv[:, base:base + DH // 2]
            to = qkv[:, base + DH // 2:base + DH]
            qkv_ref[slot, :, :DH // 2] = ((te * cos - to * sin) * sc
                                          ).astype(jnp.bfloat16)
            qkv_ref[slot, :, DH // 2:] = ((to * cos + te * sin) * sc
                                          ).astype(jnp.bfloat16)
    for h in range(H):
        base = 2 * D + h * DH
        qkv_ref[2 * H + h, :, :] = qkv[:, base:base + DH].astype(jnp.bfloat16)


def _make_att_body(qb_off, sk):
    def body(q_ref, k_ref, v_ref, *rest):
        o_ref = rest[-1]
        qb = qb_off + pl.program_id(1)
        q = q_ref[0]
        k = k_ref[0]
        s = jax.lax.dot_general(q, k, (((1,), (1,)), ((), ())),
                                preferred_element_type=jnp.float32)
        rows = qb * TB + jax.lax.broadcasted_iota(jnp.int32, (TB, sk), 0)
        cols = jax.lax.broadcasted_iota(jnp.int32, (TB, sk), 1)
        p = jnp.exp(jnp.where(cols <= rows, s, -jnp.inf))
        denom = jnp.sum(p, axis=1, keepdims=True)
        o = jnp.dot(p.astype(jnp.bfloat16), v_ref[0],
                    preferred_element_type=jnp.float32)
        o_ref[0] = o / denom
    return body


def _gate_body(a_ref, x_ref, wo_ref, ln_ref, gwt_ref, eids_ref,
               x1_ref, xn2_ref, comb_ref):
    a2 = jnp.concatenate([a_ref[h] for h in range(H)], axis=1)
    a = jnp.dot(a2.astype(jnp.bfloat16), wo_ref[...].astype(jnp.bfloat16),
                preferred_element_type=jnp.float32)

    x1 = a + x_ref[...]
    x1_ref[...] = x1
    rs = jax.lax.rsqrt(jnp.mean(x1 * x1, axis=1, keepdims=True) + EPS)
    xn2 = x1 * rs * ln_ref[...]
    xn2_ref[...] = xn2.astype(jnp.bfloat16)
    sc = jax.nn.sigmoid(jnp.dot(xn2, gwt_ref[...],
                                preferred_element_type=jnp.float32))
    e0 = eids_ref[:, 0:1]
    e1 = eids_ref[:, 1:2]
    ee = jax.lax.broadcasted_iota(jnp.int32, (TB, E), 1)
    m0 = (ee == e0)
    m1 = (ee == e1)
    w0 = jnp.sum(jnp.where(m0, sc, 0.0), axis=1, keepdims=True)
    w1_ = jnp.sum(jnp.where(m1, sc, 0.0), axis=1, keepdims=True)
    scale = ROUTE_SCALE / (w0 + w1_ + 1e-20)
    comb_ref[...] = (jnp.where(m0, w0, 0.0) + jnp.where(m1, w1_, 0.0)) * scale


def _moe_body(xn2_ref, w1_ref, w3_ref, w2_ref, comb_ref, x1_ref, out_ref):
    e = pl.program_id(0)

    @pl.when(e == 0)
    def _():
        out_ref[...] = x1_ref[...]

    xn2 = xn2_ref[...]
    h = jnp.dot(xn2, w1_ref[0].astype(jnp.bfloat16),
                preferred_element_type=jnp.float32)
    g = jnp.dot(xn2, w3_ref[0].astype(jnp.bfloat16),
                preferred_element_type=jnp.float32)
    he = (h * jax.nn.sigmoid(h) * g).astype(jnp.bfloat16)
    o = jnp.dot(he, w2_ref[0].astype(jnp.bfloat16),
                preferred_element_type=jnp.float32)
    onehot = (jax.lax.broadcasted_iota(jnp.int32, (E, 1), 0) == e
              ).astype(jnp.float32)
    ccol = jnp.dot(comb_ref[...], onehot, preferred_element_type=jnp.float32)
    out_ref[...] += ccol * o


def kernel(x, freqs_cis, input_ids, w_ln1, w_ln2, wq, wk, wv, wo, gate_w,
           tid2eid, w1, w2, w3):
    x2d = x.reshape(S, D)
    perm = jnp.concatenate([jnp.arange(0, DH, 2), jnp.arange(1, DH, 2)])
    head_perm = (jnp.arange(H)[:, None] * DH + perm[None, :]).reshape(-1)
    wq_p = wq[:, head_perm]
    wk_p = wk[:, head_perm]
    wqkv = jnp.concatenate([wq_p, wk_p, wv], axis=1).astype(jnp.bfloat16)
    cos = freqs_cis[:, :, 0]
    sin = freqs_cis[:, :, 1]

    qkv = pl.pallas_call(
        _qkv_body,
        grid=(NTB,),
        in_specs=[
            pl.BlockSpec((TB, D), lambda i: (i, 0)),
            pl.BlockSpec((D, 3 * D), lambda i: (0, 0)),
            pl.BlockSpec((1, D), lambda i: (0, 0)),
            pl.BlockSpec((TB, DH // 2), lambda i: (i, 0)),
            pl.BlockSpec((TB, DH // 2), lambda i: (i, 0)),
        ],
        out_specs=pl.BlockSpec((3 * H, TB, DH), lambda i: (0, i, 0)),
        out_shape=jax.ShapeDtypeStruct((3 * H, S, DH), jnp.bfloat16),
    )(x2d, wqkv, w_ln1.reshape(1, D), cos, sin)

    attn = pl.pallas_call(
        _make_att_body(0, S // 2),
        grid=(H, NTB // 2),
        in_specs=[
            pl.BlockSpec((1, TB, DH), lambda h, i: (h, i, 0)),
            pl.BlockSpec((1, S // 2, DH), lambda h, i: (h + H, 0, 0)),
            pl.BlockSpec((1, S // 2, DH), lambda h, i: (h + 2 * H, 0, 0)),
        ],
        out_specs=pl.BlockSpec((1, TB, DH), lambda h, i: (h, i, 0)),
        out_shape=jax.ShapeDtypeStruct((H, S, DH), jnp.float32),
    )(qkv, qkv, qkv)
    attn = pl.pallas_call(
        _make_att_body(NTB // 2, S),
        grid=(H, NTB // 2),
        in_specs=[
            pl.BlockSpec((1, TB, DH),
                         lambda h, i: (h, NTB // 2 + i, 0)),
            pl.BlockSpec((1, S, DH), lambda h, i: (h + H, 0, 0)),
            pl.BlockSpec((1, S, DH), lambda h, i: (h + 2 * H, 0, 0)),
            pl.BlockSpec((1, TB, DH),
                         lambda h, i: (h, NTB // 2 + i, 0)),
        ],
        out_specs=pl.BlockSpec((1, TB, DH),
                               lambda h, i: (h, NTB // 2 + i, 0)),
        out_shape=jax.ShapeDtypeStruct((H, S, DH), jnp.float32),
        input_output_aliases={3: 0},
    )(qkv, qkv, qkv, attn)

    eids = jnp.take(tid2eid, input_ids.reshape(-1), axis=0).astype(jnp.int32)

    x1, xn2, comb = pl.pallas_call(
        _gate_body,
        grid=(NTB,),
        in_specs=[
            pl.BlockSpec((H, TB, DH), lambda i: (0, i, 0)),
            pl.BlockSpec((TB, D), lambda i: (i, 0)),
            pl.BlockSpec((D, D), lambda i: (0, 0)),
            pl.BlockSpec((1, D), lambda i: (0, 0)),
            pl.BlockSpec((D, E), lambda i: (0, 0)),
            pl.BlockSpec((TB, TOPK), lambda i: (i, 0)),
        ],
        out_specs=[
            pl.BlockSpec((TB, D), lambda i: (i, 0)),
            pl.BlockSpec((TB, D), lambda i: (i, 0)),
            pl.BlockSpec((TB, E), lambda i: (i, 0)),
        ],
        out_shape=[
            jax.ShapeDtypeStruct((S, D), jnp.float32),
            jax.ShapeDtypeStruct((S, D), jnp.bfloat16),
            jax.ShapeDtypeStruct((S, E), jnp.float32),
        ],
    )(attn, x2d, wo, w_ln2.reshape(1, D), gate_w.T, eids)

    out = pl.pallas_call(
        _moe_body,
        grid=(E,),
        in_specs=[
            pl.BlockSpec((S, D), lambda e: (0, 0)),
            pl.BlockSpec((1, D, DFF), lambda e: (e, 0, 0)),
            pl.BlockSpec((1, D, DFF), lambda e: (e, 0, 0)),
            pl.BlockSpec((1, DFF, D), lambda e: (e, 0, 0)),
            pl.BlockSpec((S, E), lambda e: (0, 0)),
            pl.BlockSpec((S, D), lambda e: (0, 0)),
        ],
        out_specs=pl.BlockSpec((S, D), lambda e: (0, 0)),
        out_shape=jax.ShapeDtypeStruct((S, D), jnp.float32),
    )(xn2, w1, w3, w2, comb, x1)

    return out.reshape(B, S, D)


# speedup vs baseline: 1.8373x; 1.0150x over previous
import functools
import math

import jax
import jax.numpy as jnp
from jax.experimental import pallas as pl
from jax.experimental.pallas import tpu as pltpu

B, S, D = 1, 2048, 768
H, DH = 12, 64
E, TOPK, DFF = 8, 2, 256
EPS = 1e-6
ROUTE_SCALE = 2.5
TB = 256
NTB = S // TB


def _qkv_body(x_ref, wqkv_ref, ln_ref, csq_ref, snq_ref, csk_ref, snk_ref,
              qkv_ref):
    xb = x_ref[...]
    rs = jax.lax.rsqrt(jnp.mean(xb * xb, axis=1, keepdims=True) + EPS)
    xn = (xb * rs * ln_ref[...]).astype(jnp.bfloat16)
    qkv = jnp.dot(xn, wqkv_ref[...], preferred_element_type=jnp.float32)
    csq = csq_ref[...]
    snq = snq_ref[...]
    csk = csk_ref[...]
    snk = snk_ref[...]
    for h in range(H):
        for slot, base, cs, sn in ((h, h * DH, csq, snq),
                                   (H + h, D + h * DH, csk, snk)):
            t = qkv[:, base:base + DH]
            r = t * cs + jnp.roll(t, DH // 2, axis=1) * sn
            qkv_ref[slot, :, :] = r.astype(jnp.bfloat16)
    for h in range(H):
        base = 2 * D + h * DH
        qkv_ref[2 * H + h, :, :] = qkv[:, base:base + DH].astype(jnp.bfloat16)


def _make_att_body(qb_off, sk):
    def body(q_ref, k_ref, v_ref, *rest):
        o_ref = rest[-1]
        qb = qb_off + pl.program_id(1)
        q = q_ref[0]
        k = k_ref[0]
        s = jax.lax.dot_general(q, k, (((1,), (1,)), ((), ())),
                                preferred_element_type=jnp.float32)
        rows = qb * TB + jax.lax.broadcasted_iota(jnp.int32, (TB, sk), 0)
        cols = jax.lax.broadcasted_iota(jnp.int32, (TB, sk), 1)
        p = jnp.exp(jnp.where(cols <= rows, s, -jnp.inf))
        denom = jnp.sum(p, axis=1, keepdims=True)
        o = jnp.dot(p.astype(jnp.bfloat16), v_ref[0],
                    preferred_element_type=jnp.float32)
        o_ref[0] = o / denom
    return body


def _gate_body(a_ref, x_ref, wo_ref, ln_ref, gwt_ref, eids_ref,
               x1_ref, xn2_ref, comb_ref):
    a2 = jnp.concatenate([a_ref[h] for h in range(H)], axis=1)
    a = jnp.dot(a2.astype(jnp.bfloat16), wo_ref[...].astype(jnp.bfloat16),
                preferred_element_type=jnp.float32)

    x1 = a + x_ref[...]
    x1_ref[...] = x1
    rs = jax.lax.rsqrt(jnp.mean(x1 * x1, axis=1, keepdims=True) + EPS)
    xn2 = x1 * rs * ln_ref[...]
    xn2_ref[...] = xn2.astype(jnp.bfloat16)
    sc = jax.nn.sigmoid(jnp.dot(xn2, gwt_ref[...],
                                preferred_element_type=jnp.float32))
    e0 = eids_ref[:, 0:1]
    e1 = eids_ref[:, 1:2]
    ee = jax.lax.broadcasted_iota(jnp.int32, (TB, E), 1)
    m0 = (ee == e0)
    m1 = (ee == e1)
    w0 = jnp.sum(jnp.where(m0, sc, 0.0), axis=1, keepdims=True)
    w1_ = jnp.sum(jnp.where(m1, sc, 0.0), axis=1, keepdims=True)
    scale = ROUTE_SCALE / (w0 + w1_ + 1e-20)
    comb_ref[...] = (jnp.where(m0, w0, 0.0) + jnp.where(m1, w1_, 0.0)) * scale


def _moe_body(xn2_ref, w1_ref, w3_ref, w2_ref, comb_ref, x1_ref, out_ref):
    e = pl.program_id(0)

    @pl.when(e == 0)
    def _():
        out_ref[...] = x1_ref[...]

    xn2 = xn2_ref[...]
    h = jnp.dot(xn2, w1_ref[0].astype(jnp.bfloat16),
                preferred_element_type=jnp.float32)
    g = jnp.dot(xn2, w3_ref[0].astype(jnp.bfloat16),
                preferred_element_type=jnp.float32)
    he = (h * jax.nn.sigmoid(h) * g).astype(jnp.bfloat16)
    o = jnp.dot(he, w2_ref[0].astype(jnp.bfloat16),
                preferred_element_type=jnp.float32)
    onehot = (jax.lax.broadcasted_iota(jnp.int32, (E, 1), 0) == e
              ).astype(jnp.float32)
    ccol = jnp.dot(comb_ref[...], onehot, preferred_element_type=jnp.float32)
    out_ref[...] += ccol * o


def kernel(x, freqs_cis, input_ids, w_ln1, w_ln2, wq, wk, wv, wo, gate_w,
           tid2eid, w1, w2, w3):
    x2d = x.reshape(S, D)
    perm = jnp.concatenate([jnp.arange(0, DH, 2), jnp.arange(1, DH, 2)])
    head_perm = (jnp.arange(H)[:, None] * DH + perm[None, :]).reshape(-1)
    wq_p = wq[:, head_perm]
    wk_p = wk[:, head_perm]
    wqkv = jnp.concatenate([wq_p, wk_p, wv], axis=1).astype(jnp.bfloat16)
    cos = freqs_cis[:, :, 0]
    sin = freqs_cis[:, :, 1]
    cs = jnp.concatenate([cos, cos], axis=1)
    sn = jnp.concatenate([-sin, sin], axis=1)
    scale = 1.0 / math.sqrt(DH)
    csq, snq = cs * scale, sn * scale
    csk, snk = cs, sn

    qkv = pl.pallas_call(
        _qkv_body,
        grid=(NTB,),
        in_specs=[
            pl.BlockSpec((TB, D), lambda i: (i, 0)),
            pl.BlockSpec((D, 3 * D), lambda i: (0, 0)),
            pl.BlockSpec((1, D), lambda i: (0, 0)),
            pl.BlockSpec((TB, DH), lambda i: (i, 0)),
            pl.BlockSpec((TB, DH), lambda i: (i, 0)),
            pl.BlockSpec((TB, DH), lambda i: (i, 0)),
            pl.BlockSpec((TB, DH), lambda i: (i, 0)),
        ],
        out_specs=pl.BlockSpec((3 * H, TB, DH), lambda i: (0, i, 0)),
        out_shape=jax.ShapeDtypeStruct((3 * H, S, DH), jnp.bfloat16),
    )(x2d, wqkv, w_ln1.reshape(1, D), csq, snq, csk, snk)

    attn = pl.pallas_call(
        _make_att_body(0, S // 2),
        grid=(H, NTB // 2),
        in_specs=[
            pl.BlockSpec((1, TB, DH), lambda h, i: (h, i, 0)),
            pl.BlockSpec((1, S // 2, DH), lambda h, i: (h + H, 0, 0)),
            pl.BlockSpec((1, S // 2, DH), lambda h, i: (h + 2 * H, 0, 0)),
        ],
        out_specs=pl.BlockSpec((1, TB, DH), lambda h, i: (h, i, 0)),
        out_shape=jax.ShapeDtypeStruct((H, S, DH), jnp.float32),
    )(qkv, qkv, qkv)
    attn = pl.pallas_call(
        _make_att_body(NTB // 2, S),
        grid=(H, NTB // 2),
        in_specs=[
            pl.BlockSpec((1, TB, DH),
                         lambda h, i: (h, NTB // 2 + i, 0)),
            pl.BlockSpec((1, S, DH), lambda h, i: (h + H, 0, 0)),
            pl.BlockSpec((1, S, DH), lambda h, i: (h + 2 * H, 0, 0)),
            pl.BlockSpec((1, TB, DH),
                         lambda h, i: (h, NTB // 2 + i, 0)),
        ],
        out_specs=pl.BlockSpec((1, TB, DH),
                               lambda h, i: (h, NTB // 2 + i, 0)),
        out_shape=jax.ShapeDtypeStruct((H, S, DH), jnp.float32),
        input_output_aliases={3: 0},
    )(qkv, qkv, qkv, attn)

    eids = jnp.take(tid2eid, input_ids.reshape(-1), axis=0).astype(jnp.int32)

    x1, xn2, comb = pl.pallas_call(
        _gate_body,
        grid=(NTB,),
        in_specs=[
            pl.BlockSpec((H, TB, DH), lambda i: (0, i, 0)),
            pl.BlockSpec((TB, D), lambda i: (i, 0)),
            pl.BlockSpec((D, D), lambda i: (0, 0)),
            pl.BlockSpec((1, D), lambda i: (0, 0)),
            pl.BlockSpec((D, E), lambda i: (0, 0)),
            pl.BlockSpec((TB, TOPK), lambda i: (i, 0)),
        ],
        out_specs=[
            pl.BlockSpec((TB, D), lambda i: (i, 0)),
            pl.BlockSpec((TB, D), lambda i: (i, 0)),
            pl.BlockSpec((TB, E), lambda i: (i, 0)),
        ],
        out_shape=[
            jax.ShapeDtypeStruct((S, D), jnp.float32),
            jax.ShapeDtypeStruct((S, D), jnp.bfloat16),
            jax.ShapeDtypeStruct((S, E), jnp.float32),
        ],
    )(attn, x2d, wo, w_ln2.reshape(1, D), gate_w.T, eids)

    out = pl.pallas_call(
        _moe_body,
        grid=(E,),
        in_specs=[
            pl.BlockSpec((S, D), lambda e: (0, 0)),
            pl.BlockSpec((1, D, DFF), lambda e: (e, 0, 0)),
            pl.BlockSpec((1, D, DFF), lambda e: (e, 0, 0)),
            pl.BlockSpec((1, DFF, D), lambda e: (e, 0, 0)),
            pl.BlockSpec((S, E), lambda e: (0, 0)),
            pl.BlockSpec((S, D), lambda e: (0, 0)),
        ],
        out_specs=pl.BlockSpec((S, D), lambda e: (0, 0)),
        out_shape=jax.ShapeDtypeStruct((S, D), jnp.float32),
    )(xn2, w1, w3, w2, comb, x1)

    return out.reshape(B, S, D)
